# baseline (speedup 1.0000x reference)
import functools

import jax
import jax.numpy as jnp
from jax import lax
from jax.experimental import pallas as pl
from jax.experimental.pallas import tpu as pltpu
from jax.experimental.pallas import tpu_sc as plsc

NC, NS, L = 2, 16, 16
NW = NC * NS

BN = 2048


def _leaky(x):
    return jnp.maximum(x, 0.01 * x)


def _sc_gather_rows(table, act):
    _, ED = table.shape
    (B,) = act.shape
    rpw = B // NW

    mesh = plsc.VectorSubcoreMesh(
        core_axis_name="c", subcore_axis_name="s", num_cores=NC, num_subcores=NS
    )

    @functools.partial(
        pl.kernel,
        out_type=jax.ShapeDtypeStruct((B, ED), jnp.float32),
        mesh=mesh,
        scratch_types=[
            pltpu.VMEM((rpw,), jnp.int32),
            pltpu.VMEM((rpw, ED), jnp.float32),
            pltpu.SemaphoreType.DMA,
        ],
        compiler_params=pltpu.CompilerParams(use_tc_tiling_on_sc=True),
    )
    def gather_kernel(table_hbm, act_hbm, out_hbm, idx_v, emb_v, sem):
        wid = lax.axis_index("s") * NC + lax.axis_index("c")
        b0 = wid * rpw
        pltpu.sync_copy(act_hbm.at[pl.ds(b0, rpw)], idx_v)

        def fire(g, carry):
            vec = idx_v[pl.ds(g * L, L)]
            for lane in range(L):
                r = vec[lane]
                pltpu.async_copy(table_hbm.at[r], emb_v.at[g * L + lane], sem)
            return carry

        lax.fori_loop(0, rpw // L, fire, 0)
        pltpu.make_async_copy(table_hbm.at[pl.ds(0, rpw)], emb_v, sem).wait()
        pltpu.sync_copy(emb_v, out_hbm.at[pl.ds(b0, rpw)])

    return gather_kernel(table, act)


def _tc_assemble(state, emb2_raw, parity):
    B, SD = state.shape
    _, ED2 = emb2_raw.shape
    ED = ED2 // 2
    OD = SD + ED

    def body(st_ref, emb_ref, par_ref, out_ref):
        out_ref[pl.ds(0, SD), :] = _leaky(st_ref[...].T)
        e2t = emb_ref[...].T
        pmask = par_ref[...] != 0
        sel = jnp.where(pmask, e2t[ED:, :], e2t[:ED, :])
        out_ref[pl.ds(SD, ED), :] = _leaky(sel)

    return pl.pallas_call(
        body,
        grid=(B // BN,),
        in_specs=[
            pl.BlockSpec((BN, SD), lambda i: (i, 0)),
            pl.BlockSpec((BN, ED2), lambda i: (i, 0)),
            pl.BlockSpec((1, BN), lambda i: (0, i)),
        ],
        out_specs=pl.BlockSpec((OD, BN), lambda i: (0, i)),
        out_shape=jax.ShapeDtypeStruct((OD, B), jnp.float32),
    )(state, emb2_raw, parity)


def kernel(state, action, emb_table):
    act = action.astype(jnp.int32)
    table2 = emb_table.reshape(-1, 2 * emb_table.shape[1])
    emb2_raw = _sc_gather_rows(table2, act >> 1)
    out_t = _tc_assemble(state, emb2_raw, (act & 1).reshape(1, -1))
    return out_t.T
